# Initial kernel scaffold; baseline (speedup 1.0000x reference)
import jax
import jax.numpy as jnp
from jax.experimental import pallas as pl


def kernel(nfeats, efeats, edge_index, W_apply_w, W_apply_b):
    raise NotImplementedError("write your pallas kernel here")


# speedup vs baseline: 6.4459x; 6.4459x over previous
import jax
import jax.numpy as jnp
from jax import lax
from jax.experimental import pallas as pl
from jax.experimental.pallas import tpu as pltpu
from jax.experimental.pallas import tpu_sc as plsc

N_NODES = 10000
N_EDGES = 320000
D_IN = 128
E_DIM = 16
D_OUT = 128

NUM_CORES = 2
NUM_SUBCORES = 16
NW = NUM_CORES * NUM_SUBCORES
E_PER_W = N_EDGES // NW
CHUNK = 2000
N_CHUNKS = E_PER_W // CHUNK
IB = 100
IB_PER_CHUNK = CHUNK // IB
IB_PER_W = E_PER_W // IB
N_PAD = 10240
ROWS_PER_SUB = N_PAD // NUM_SUBCORES


def _sc_aggregate(ef2d, dst3d, zeros_in, zeros1_in):
  mesh = plsc.VectorSubcoreMesh(core_axis_name="c", subcore_axis_name="s",
                                num_cores=NUM_CORES,
                                num_subcores=NUM_SUBCORES)

  @pl.kernel(
      out_type=(
          jax.ShapeDtypeStruct((NUM_CORES, N_PAD, E_DIM), jnp.float32),
          jax.ShapeDtypeStruct((NUM_CORES, N_PAD), jnp.float32),
      ),
      mesh=mesh,
      compiler_params=pltpu.CompilerParams(use_tc_tiling_on_sc=False),
      scratch_types=[
          pltpu.VMEM_SHARED((N_PAD, E_DIM), jnp.float32),
          pltpu.VMEM_SHARED((N_PAD,), jnp.float32),
          pltpu.VMEM((CHUNK, E_DIM), jnp.float32),
          pltpu.VMEM((IB_PER_W, IB), jnp.int32),
          pltpu.VMEM((112,), jnp.float32),
      ],
  )
  def k(ef_hbm, dst_hbm, zeros_hbm, zeros1_hbm, hs_hbm, dg_hbm,
        acc, dacc, rows_v, idx_v, ones_v):
    c = lax.axis_index("c")
    s = lax.axis_index("s")
    wid = c * NUM_SUBCORES + s

    r0 = s * ROWS_PER_SUB
    pltpu.sync_copy(zeros_hbm.at[pl.ds(0, ROWS_PER_SUB), :],
                    acc.at[pl.ds(r0, ROWS_PER_SUB), :])
    pltpu.sync_copy(zeros1_hbm, dacc.at[pl.ds(r0, ROWS_PER_SUB)])

    for i in range(7):
      ones_v[pl.ds(i * 16, 16)] = jnp.full((16,), 1.0, jnp.float32)
    pltpu.sync_copy(dst_hbm.at[wid], idx_v)
    plsc.subcore_barrier()

    @pl.loop(0, N_CHUNKS)
    def _(kk):
      base = wid * E_PER_W + kk * CHUNK
      pltpu.sync_copy(ef_hbm.at[pl.ds(base, CHUNK), :], rows_v)

      @pl.loop(0, IB_PER_CHUNK)
      def _(j):
        idx = idx_v.at[kk * IB_PER_CHUNK + j]
        pltpu.sync_copy(rows_v.at[pl.ds(j * IB, IB), :], acc.at[idx],
                        add=True)
        pltpu.sync_copy(ones_v.at[pl.ds(0, IB)], dacc.at[idx], add=True)

    plsc.subcore_barrier()

    pltpu.sync_copy(acc.at[pl.ds(r0, ROWS_PER_SUB), :],
                    hs_hbm.at[c, pl.ds(r0, ROWS_PER_SUB), :])
    pltpu.sync_copy(dacc.at[pl.ds(r0, ROWS_PER_SUB)],
                    dg_hbm.at[c, pl.ds(r0, ROWS_PER_SUB)])

  return k(ef2d, dst3d, zeros_in, zeros1_in)


ROW_BLK = 1000


def _tc_body(nf_ref, hs_ref, dg_ref, wn_ref, we_ref, b_ref, out_ref):
  hs = hs_ref[0] + hs_ref[1]
  dg = dg_ref[0] + dg_ref[1]
  hn = hs / jnp.maximum(dg, 1.0)
  acc = jnp.dot(nf_ref[...], wn_ref[...],
                preferred_element_type=jnp.float32)
  acc = acc + jnp.dot(hn, we_ref[...],
                      preferred_element_type=jnp.float32)
  out_ref[...] = jnp.maximum(acc + b_ref[...], 0.0)


def _tc_apply(nf2d, hsum, deg, wn, we, bias2d):
  grid = (N_NODES // ROW_BLK,)
  return pl.pallas_call(
      _tc_body,
      grid=grid,
      in_specs=[
          pl.BlockSpec((ROW_BLK, D_IN), lambda i: (i, 0)),
          pl.BlockSpec((NUM_CORES, ROW_BLK, E_DIM), lambda i: (0, i, 0)),
          pl.BlockSpec((NUM_CORES, ROW_BLK, 1), lambda i: (0, i, 0)),
          pl.BlockSpec((D_IN, D_OUT), lambda i: (0, 0)),
          pl.BlockSpec((E_DIM, D_OUT), lambda i: (0, 0)),
          pl.BlockSpec((1, D_OUT), lambda i: (0, 0)),
      ],
      out_specs=pl.BlockSpec((ROW_BLK, D_OUT), lambda i: (i, 0)),
      out_shape=jax.ShapeDtypeStruct((N_NODES, D_OUT), jnp.float32),
  )(nf2d, hsum, deg, wn, we, bias2d)


def kernel(nfeats, efeats, edge_index, W_apply_w, W_apply_b):
  nf2d = nfeats.reshape(N_NODES, D_IN)
  ef2d = efeats.reshape(N_EDGES, E_DIM)
  dst3d = edge_index[1].reshape(NW, IB_PER_W, IB)
  zeros_in = jnp.zeros((ROWS_PER_SUB, E_DIM), jnp.float32)
  zeros1_in = jnp.zeros((ROWS_PER_SUB,), jnp.float32)

  hsum, deg = _sc_aggregate(ef2d, dst3d, zeros_in, zeros1_in)
  deg = deg.reshape(NUM_CORES, N_PAD, 1)

  wn = W_apply_w[:, :D_IN].T
  we = W_apply_w[:, D_IN:].T
  bias2d = W_apply_b.reshape(1, D_OUT)
  out2d = _tc_apply(nf2d, hsum, deg, wn, we, bias2d)
  return out2d.reshape(N_NODES, 1, D_OUT)


# speedup vs baseline: 8.7105x; 1.3513x over previous
import jax
import jax.numpy as jnp
from jax import lax
from jax.experimental import pallas as pl
from jax.experimental.pallas import tpu as pltpu
from jax.experimental.pallas import tpu_sc as plsc

N_NODES = 10000
N_EDGES = 320000
D_IN = 128
E_DIM = 16
D_OUT = 128

NUM_CORES = 2
NUM_SUBCORES = 16
NW = NUM_CORES * NUM_SUBCORES
E_PER_W = N_EDGES // NW
CHUNK = 2000
N_CHUNKS = E_PER_W // CHUNK
IB = 100
IB_PER_CHUNK = CHUNK // IB
IB_PER_W = E_PER_W // IB
N_PAD = 10240
ROWS_PER_SUB = N_PAD // NUM_SUBCORES


def _sc_aggregate(ef2d, dst3d, zeros_in, zeros1_in):
  mesh = plsc.VectorSubcoreMesh(core_axis_name="c", subcore_axis_name="s",
                                num_cores=NUM_CORES,
                                num_subcores=NUM_SUBCORES)

  @pl.kernel(
      out_type=(
          jax.ShapeDtypeStruct((NUM_CORES, N_PAD, E_DIM), jnp.float32),
          jax.ShapeDtypeStruct((NUM_CORES, N_PAD), jnp.float32),
      ),
      mesh=mesh,
      compiler_params=pltpu.CompilerParams(use_tc_tiling_on_sc=False,
                                           needs_layout_passes=False),
      scratch_types=[
          pltpu.VMEM_SHARED((N_PAD, E_DIM), jnp.float32),
          pltpu.VMEM_SHARED((N_PAD,), jnp.float32),
          pltpu.VMEM((E_DIM, CHUNK), jnp.float32),
          pltpu.VMEM((CHUNK, E_DIM), jnp.float32),
          pltpu.VMEM((IB_PER_W, IB), jnp.int32),
          pltpu.VMEM((112,), jnp.float32),
      ],
  )
  def k(ef_hbm, dst_hbm, zeros_hbm, zeros1_hbm, hs_hbm, dg_hbm,
        acc, dacc, cols_v, rows_v, idx_v, ones_v):
    c = lax.axis_index("c")
    s = lax.axis_index("s")
    wid = c * NUM_SUBCORES + s

    r0 = s * ROWS_PER_SUB
    pltpu.sync_copy(zeros_hbm.at[pl.ds(0, ROWS_PER_SUB), :],
                    acc.at[pl.ds(r0, ROWS_PER_SUB), :])
    pltpu.sync_copy(zeros1_hbm, dacc.at[pl.ds(r0, ROWS_PER_SUB)])

    for i in range(7):
      ones_v[pl.ds(i * 16, 16)] = jnp.full((16,), 1.0, jnp.float32)
    pltpu.sync_copy(dst_hbm.at[wid], idx_v)
    plsc.subcore_barrier()

    lane = jax.lax.iota(jnp.int32, 16)

    @pl.loop(0, N_CHUNKS)
    def _(kk):
      base = wid * E_PER_W + kk * CHUNK
      pltpu.sync_copy(ef_hbm.at[:, pl.ds(base, CHUNK)], cols_v)

      @pl.loop(0, CHUNK // 16)
      def _(g):
        row_idx = g * 16 + lane
        for f in range(E_DIM):
          v = cols_v[f, pl.ds(g * 16, 16)]
          plsc.store_scatter(rows_v, [row_idx, jnp.full((16,), f, jnp.int32)],
                             v)

      @pl.loop(0, IB_PER_CHUNK)
      def _(j):
        idx = idx_v.at[kk * IB_PER_CHUNK + j]
        pltpu.sync_copy(rows_v.at[pl.ds(j * IB, IB), :], acc.at[idx],
                        add=True)
        pltpu.sync_copy(ones_v.at[pl.ds(0, IB)], dacc.at[idx], add=True)

    plsc.subcore_barrier()

    pltpu.sync_copy(acc.at[pl.ds(r0, ROWS_PER_SUB), :],
                    hs_hbm.at[c, pl.ds(r0, ROWS_PER_SUB), :])
    pltpu.sync_copy(dacc.at[pl.ds(r0, ROWS_PER_SUB)],
                    dg_hbm.at[c, pl.ds(r0, ROWS_PER_SUB)])

  return k(ef2d, dst3d, zeros_in, zeros1_in)


ROW_BLK = 1000


def _tc_body(nf_ref, hs_ref, dg_ref, wn_ref, we_ref, b_ref, out_ref):
  hs = hs_ref[0] + hs_ref[1]
  dg = dg_ref[0] + dg_ref[1]
  hn = hs / jnp.maximum(dg, 1.0)
  acc = jnp.dot(nf_ref[...], wn_ref[...],
                preferred_element_type=jnp.float32)
  acc = acc + jnp.dot(hn, we_ref[...],
                      preferred_element_type=jnp.float32)
  out_ref[...] = jnp.maximum(acc + b_ref[...], 0.0)


def _tc_apply(nf2d, hsum, deg, wn, we, bias2d):
  grid = (N_NODES // ROW_BLK,)
  return pl.pallas_call(
      _tc_body,
      grid=grid,
      in_specs=[
          pl.BlockSpec((ROW_BLK, D_IN), lambda i: (i, 0)),
          pl.BlockSpec((NUM_CORES, ROW_BLK, E_DIM), lambda i: (0, i, 0)),
          pl.BlockSpec((NUM_CORES, ROW_BLK, 1), lambda i: (0, i, 0)),
          pl.BlockSpec((D_IN, D_OUT), lambda i: (0, 0)),
          pl.BlockSpec((E_DIM, D_OUT), lambda i: (0, 0)),
          pl.BlockSpec((1, D_OUT), lambda i: (0, 0)),
      ],
      out_specs=pl.BlockSpec((ROW_BLK, D_OUT), lambda i: (i, 0)),
      out_shape=jax.ShapeDtypeStruct((N_NODES, D_OUT), jnp.float32),
  )(nf2d, hsum, deg, wn, we, bias2d)


def kernel(nfeats, efeats, edge_index, W_apply_w, W_apply_b):
  nf2d = nfeats.reshape(N_NODES, D_IN)
  efT = jnp.transpose(efeats[:, 0, :])
  dst3d = edge_index[1].reshape(NW, IB_PER_W, IB)
  zeros_in = jnp.zeros((ROWS_PER_SUB, E_DIM), jnp.float32)
  zeros1_in = jnp.zeros((ROWS_PER_SUB,), jnp.float32)

  hsum, deg = _sc_aggregate(efT, dst3d, zeros_in, zeros1_in)
  deg = deg.reshape(NUM_CORES, N_PAD, 1)

  wn = W_apply_w[:, :D_IN].T
  we = W_apply_w[:, D_IN:].T
  bias2d = W_apply_b.reshape(1, D_OUT)
  out2d = _tc_apply(nf2d, hsum, deg, wn, we, bias2d)
  return out2d.reshape(N_NODES, 1, D_OUT)
